# baseline (speedup 1.0000x reference)
import functools

import jax
import jax.numpy as jnp
from jax.experimental import pallas as pl
from jax.experimental.pallas import tpu as pltpu

NN = 1024
BS = 4 * NN
F0 = 64
H = 128
NC = 6


def _mega_kernel(thr_ref, c_ref, x_ref,
                 wac1_ref, bac1_ref, wca1_ref, bca1_ref,
                 wac2_ref, bac2_ref, wca2_ref, bca2_ref,
                 g1_ref, b1_ref, g2_ref, b2_ref, wout_ref,
                 out_ref, adj_scr, h1_scr, h2_scr, degr_scr, degc_scr):
    f32 = jnp.float32
    bf16 = jnp.bfloat16
    t = pl.program_id(0)

    def blk(ref, i, j, w=NN):
        return ref[i * NN:(i + 1) * NN, j * w:(j + 1) * w]

    def corr(a, b):
        return jax.lax.dot_general(a, b, (((1,), (1,)), ((), ())),
                                   preferred_element_type=f32)

    @pl.when(t == 0)
    def _():
        x = x_ref[...]
        xc = x - jnp.mean(x, axis=1, keepdims=True)
        h1_scr[:, 0:F0] = xc * jax.lax.rsqrt(jnp.sum(xc * xc, axis=1,
                                                     keepdims=True))

    rows = jax.lax.broadcasted_iota(jnp.int32, (NN, NN), 0)
    cols = jax.lax.broadcasted_iota(jnp.int32, (NN, NN), 1)

    def xnb(k):
        return h1_scr[k * NN:(k + 1) * NN, 0:F0]

    def adj_cell(i, j):
        if i == j:
            g = corr(xnb(i), xnb(i))
            noteye = (rows != cols).astype(bf16)
            ind = (jnp.abs(g) > thr_ref[0, i]).astype(bf16) * noteye
        else:
            if i > j:
                p = corr(xnb(i - 1), xnb(3))
                th = thr_ref[0, 4 + i]
            else:
                p = corr(xnb(3), xnb(j - 1))
                th = thr_ref[0, 4 + j]
            c = (jnp.abs(p) > th).astype(bf16)
            m_a = blk(adj_scr, i, i)
            cnt = jax.lax.dot_general(m_a, c, (((1,), (0,)), ((), ())),
                                      preferred_element_type=f32)
            ind = (cnt > 0.0).astype(bf16)
        adj_scr[i * NN:(i + 1) * NN, j * NN:(j + 1) * NN] = ind

    cells = [(i, i) for i in range(4)]
    cells += [(i, j) for i in range(4) for j in range(4) if i != j]
    for step, (i, j) in enumerate(cells):
        pl.when(t == 1 + step)(lambda i=i, j=j: adj_cell(i, j))

    @pl.when(t == 17)
    def _():
        ones_b = jnp.ones((NN, 1), bf16)
        for i in range(4):
            dr = jnp.zeros((NN, 1), f32)
            dc = jnp.zeros((NN, 1), f32)
            for j in range(4):
                dr = dr + jax.lax.dot_general(
                    blk(adj_scr, i, j), ones_b, (((1,), (0,)), ((), ())),
                    preferred_element_type=f32)
                dc = dc + jax.lax.dot_general(
                    blk(adj_scr, j, i), ones_b, (((0,), (0,)), ((), ())),
                    preferred_element_type=f32)
            degr_scr[i * NN:(i + 1) * NN, :] = dr
            degc_scr[i * NN:(i + 1) * NN, :] = dc

    def norms(deg):
        n = jnp.where(deg > 0, jax.lax.rsqrt(jnp.maximum(deg, 1.0)), 0.0)
        return n, n / jnp.maximum(deg, 1.0)

    def gin_step(d, xin_at, wac, bac, wca, bca, h_out):
        n_c, sc_c = norms(degc_scr[...])
        n_r, sc_r = norms(degr_scr[...])
        sl = slice(d * NN, (d + 1) * NN)
        nf = None
        agg_ac = None
        agg_ca = None
        for s in range(4):
            ss = slice(s * NN, (s + 1) * NN)
            x_s = xin_at(s)
            u_s = (x_s * n_c[ss]).astype(bf16)
            v_s = (x_s * n_r[ss]).astype(bf16)
            a = jax.lax.dot_general(adj_scr[ss, sl], u_s,
                                    (((0,), (0,)), ((), ())),
                                    preferred_element_type=f32)
            b = jax.lax.dot_general(adj_scr[sl, ss], v_s,
                                    (((1,), (0,)), ((), ())),
                                    preferred_element_type=f32)
            agg_ac = a if agg_ac is None else agg_ac + a
            agg_ca = b if agg_ca is None else agg_ca + b
        x_d = xin_at(d)
        agg_ac = agg_ac * sc_c[sl]
        agg_ca = agg_ca * sc_r[sl]
        z_ac = jax.nn.relu(
            jnp.dot(x_d + agg_ac, wac, preferred_element_type=f32) + bac)
        z_ca = jax.nn.relu(
            jnp.dot(x_d + agg_ca, wca, preferred_element_type=f32) + bca)
        h_out(d, jnp.concatenate([z_ac, z_ca], axis=1))

    def l1_step(d):
        gin_step(d,
                 lambda s: x_ref[s * NN:(s + 1) * NN, :],
                 wac1_ref[...], bac1_ref[...], wca1_ref[...], bca1_ref[...],
                 lambda d2, h: h1_scr.__setitem__(
                     (slice(d2 * NN, (d2 + 1) * NN), slice(None)), h))
    for d in range(4):
        pl.when(t == 18 + d)(lambda d=d: l1_step(d))

    def bn_stats(h_scr):
        mu = jnp.zeros((1, 2 * H), f32)
        msq = jnp.zeros((1, 2 * H), f32)
        for s in range(4):
            h_s = h_scr[s * NN:(s + 1) * NN, :]
            mu = mu + jnp.sum(h_s, axis=0, keepdims=True)
            msq = msq + jnp.sum(h_s * h_s, axis=0, keepdims=True)
        mu = mu / BS
        var = msq / BS - mu * mu
        return mu, var

    def l2_step(d):
        mu, var = bn_stats(h1_scr)
        scale = jax.lax.rsqrt(var + 1e-5) * g1_ref[...]
        bias = b1_ref[...]
        gin_step(d,
                 lambda s: (h1_scr[s * NN:(s + 1) * NN, :] - mu) * scale + bias,
                 wac2_ref[...], bac2_ref[...], wca2_ref[...], bca2_ref[...],
                 lambda d2, h: h2_scr.__setitem__(
                     (slice(d2 * NN, (d2 + 1) * NN), slice(None)), h))
    for d in range(4):
        pl.when(t == 22 + d)(lambda d=d: l2_step(d))

    @pl.when(t == 26)
    def _():
        mu, var = bn_stats(h2_scr)
        scale = jax.lax.rsqrt(var + 1e-5) * g2_ref[...]
        bias = b2_ref[...]
        c0 = c_ref[0, 0]
        c1 = c_ref[0, 1]
        c2 = c_ref[0, 2]
        c3 = c_ref[0, 3]
        m = jnp.maximum(jnp.maximum(c0, c1), jnp.maximum(c2, c3))
        e0 = jnp.exp(c0 - m)
        e1 = jnp.exp(c1 - m)
        e2 = jnp.exp(c2 - m)
        e3 = jnp.exp(c3 - m)
        den = e0 + e1 + e2 + e3
        acc = None
        for s, w in enumerate([e0 / den, e1 / den, e2 / den, e3 / den]):
            x3 = (h2_scr[s * NN:(s + 1) * NN, :] - mu) * scale + bias
            y = jnp.dot(x3, wout_ref[...], preferred_element_type=f32) * w
            acc = y if acc is None else acc + y
        out_ref[...] = acc


@functools.partial(jax.jit, static_argnames=())
def kernel(features, sparse, c_param, W_ac1, b_ac1, W_ca1, b_ca1,
           W_ac2, b_ac2, W_ca2, b_ca2, bn1_g, bn1_b, bn2_g, bn2_b, W_out):
    f32 = jnp.float32

    sig = jax.nn.sigmoid(sparse[:, 0])
    thr = jnp.stack([sig[1], sig[5], sig[8], sig[10],
                     jnp.float32(0.0), sig[4], sig[7], sig[9]])[None, :]

    b2 = lambda a: a[None, :]
    vmem = lambda shape: pl.BlockSpec(shape, lambda t: tuple(0 for _ in shape))

    out = pl.pallas_call(
        _mega_kernel,
        grid=(27,),
        in_specs=[
            pl.BlockSpec(memory_space=pltpu.SMEM),
            pl.BlockSpec(memory_space=pltpu.SMEM),
            vmem((BS, F0)),
            vmem((F0, H)), vmem((1, H)), vmem((F0, H)), vmem((1, H)),
            vmem((2 * H, H)), vmem((1, H)), vmem((2 * H, H)), vmem((1, H)),
            vmem((1, 2 * H)), vmem((1, 2 * H)),
            vmem((1, 2 * H)), vmem((1, 2 * H)),
            vmem((2 * H, NC)),
        ],
        out_specs=vmem((NN, NC)),
        out_shape=jax.ShapeDtypeStruct((NN, NC), f32),
        scratch_shapes=[
            pltpu.VMEM((BS, BS), jnp.bfloat16),
            pltpu.VMEM((BS, 2 * H), f32),
            pltpu.VMEM((BS, 2 * H), f32),
            pltpu.VMEM((BS, 1), f32),
            pltpu.VMEM((BS, 1), f32),
        ],
    )(thr, c_param, features,
      W_ac1, b2(b_ac1), W_ca1, b2(b_ca1),
      W_ac2, b2(b_ac2), W_ca2, b2(b_ca2),
      b2(bn1_g), b2(bn1_b), b2(bn2_g), b2(bn2_b), W_out)

    return out


# speedup vs baseline: 9.8897x; 9.8897x over previous
import functools

import jax
import jax.numpy as jnp
from jax.experimental import pallas as pl
from jax.experimental.pallas import tpu as pltpu

NN = 1024
BS = 4 * NN
F0 = 64
H = 128
NC = 6


def _norm_kernel(x_ref, out_ref):
    x = x_ref[...]
    xc = x - jnp.mean(x, axis=1, keepdims=True)
    out_ref[...] = xc * jax.lax.rsqrt(jnp.sum(xc * xc, axis=1, keepdims=True))


def _adj_kernel(thr_ref, xni_ref, xna_ref, xnb_ref, eye_ref, noteye_ref,
                adj_ref, pr_ref, pc_ref):
    i = pl.program_id(0)
    j = pl.program_id(1)

    xn_i = xni_ref[...]
    g_ii = jax.lax.dot_general(xn_i, xn_i, (((1,), (1,)), ((), ())),
                               preferred_element_type=jnp.float32)
    mask_a = (jnp.abs(g_ii) > thr_ref[0, i]).astype(jnp.bfloat16) * noteye_ref[...]

    p = jax.lax.dot_general(xna_ref[...], xnb_ref[...], (((1,), (1,)), ((), ())),
                            preferred_element_type=jnp.float32)
    th_c = thr_ref[0, 4 + jnp.maximum(i, j)]
    mask_c = jnp.where(i == j, eye_ref[...],
                       (jnp.abs(p) > th_c).astype(jnp.bfloat16))

    cnt = jax.lax.dot_general(mask_a, mask_c,
                              (((1,), (0,)), ((), ())),
                              preferred_element_type=jnp.float32)
    ind = (cnt > 0.0).astype(jnp.float32)
    adj_ref[...] = ind.astype(jnp.int8)
    ones = jnp.ones((NN, 1), jnp.float32)
    pr_ref[...] = jnp.sum(ind, axis=1, keepdims=True)[None]
    pc_ref[...] = jax.lax.dot_general(ind, ones, (((0,), (0,)), ((), ())),
                                      preferred_element_type=jnp.float32)[None]


def _degsum_kernel(pr_ref, pc_ref, nr_ref, nc_ref, scr_ref, scc_ref):
    degr = jnp.sum(pr_ref[...], axis=0)
    degc = jnp.sum(pc_ref[...], axis=0)
    n_r = jnp.where(degr > 0, jax.lax.rsqrt(jnp.maximum(degr, 1.0)), 0.0)
    n_c = jnp.where(degc > 0, jax.lax.rsqrt(jnp.maximum(degc, 1.0)), 0.0)
    nr_ref[...] = n_r
    nc_ref[...] = n_c
    scr_ref[...] = n_r / jnp.maximum(degr, 1.0)
    scc_ref[...] = n_c / jnp.maximum(degc, 1.0)


def _gin_block(adj_col, adj_row, x, x_d, n_c, n_r, scc_d, scr_d,
               w_ac, b_ac, w_ca, b_ca):
    u = (x * n_c).astype(jnp.bfloat16)
    v = (x * n_r).astype(jnp.bfloat16)
    agg_ac = jax.lax.dot_general(adj_col.astype(jnp.bfloat16), u,
                                 (((0,), (0,)), ((), ())),
                                 preferred_element_type=jnp.float32)
    agg_ca = jax.lax.dot_general(adj_row.astype(jnp.bfloat16), v,
                                 (((1,), (0,)), ((), ())),
                                 preferred_element_type=jnp.float32)
    agg_ac = agg_ac * scc_d
    agg_ca = agg_ca * scr_d
    z_ac = jax.nn.relu(
        jnp.dot(x_d + agg_ac, w_ac, preferred_element_type=jnp.float32) + b_ac)
    z_ca = jax.nn.relu(
        jnp.dot(x_d + agg_ca, w_ca, preferred_element_type=jnp.float32) + b_ca)
    return jnp.concatenate([z_ac, z_ca], axis=1)


def _l1_kernel(adj_col_ref, adj_row_ref, x_ref, xd_ref, nr_ref, nc_ref,
               scrd_ref, sccd_ref,
               wac_ref, bac_ref, wca_ref, bca_ref,
               h_ref, ss_ref, sq_ref):
    h_d = _gin_block(adj_col_ref[...], adj_row_ref[...], x_ref[...],
                     xd_ref[...], nc_ref[...], nr_ref[...],
                     sccd_ref[...], scrd_ref[...],
                     wac_ref[...], bac_ref[...], wca_ref[...], bca_ref[...])
    h_ref[...] = h_d
    ss_ref[...] = jnp.sum(h_d, axis=0, keepdims=True)[None]
    sq_ref[...] = jnp.sum(h_d * h_d, axis=0, keepdims=True)[None]


def _l2_kernel(adj_col_ref, adj_row_ref, h1_ref, h1d_ref, ss_ref, sq_ref,
               g_ref, b_ref, nr_ref, nc_ref, scrd_ref, sccd_ref,
               wac_ref, bac_ref, wca_ref, bca_ref,
               h_ref, ss2_ref, sq2_ref):
    mu = jnp.sum(ss_ref[...][:, 0, :], axis=0, keepdims=True) / BS
    msq = jnp.sum(sq_ref[...][:, 0, :], axis=0, keepdims=True) / BS
    var = msq - mu * mu
    scale = jax.lax.rsqrt(var + 1e-5) * g_ref[...]
    bias = b_ref[...]
    x = (h1_ref[...] - mu) * scale + bias
    x_d = (h1d_ref[...] - mu) * scale + bias
    h_d = _gin_block(adj_col_ref[...], adj_row_ref[...], x, x_d,
                     nc_ref[...], nr_ref[...], sccd_ref[...], scrd_ref[...],
                     wac_ref[...], bac_ref[...], wca_ref[...], bca_ref[...])
    h_ref[...] = h_d
    ss2_ref[...] = jnp.sum(h_d, axis=0, keepdims=True)[None]
    sq2_ref[...] = jnp.sum(h_d * h_d, axis=0, keepdims=True)[None]


def _out_kernel(c_ref, h2_ref, ss_ref, sq_ref, g_ref, b_ref, wout_ref,
                out_ref):
    mu = jnp.sum(ss_ref[...][:, 0, :], axis=0, keepdims=True) / BS
    msq = jnp.sum(sq_ref[...][:, 0, :], axis=0, keepdims=True) / BS
    var = msq - mu * mu
    scale = jax.lax.rsqrt(var + 1e-5) * g_ref[...]
    h = (h2_ref[...] - mu) * scale + b_ref[...]
    y = jnp.dot(h, wout_ref[...], preferred_element_type=jnp.float32)
    c0 = c_ref[0, 0]
    c1 = c_ref[0, 1]
    c2 = c_ref[0, 2]
    c3 = c_ref[0, 3]
    m = jnp.maximum(jnp.maximum(c0, c1), jnp.maximum(c2, c3))
    e0 = jnp.exp(c0 - m)
    e1 = jnp.exp(c1 - m)
    e2 = jnp.exp(c2 - m)
    e3 = jnp.exp(c3 - m)
    den = e0 + e1 + e2 + e3
    out_ref[...] = (y[0 * NN:1 * NN] * (e0 / den) +
                    y[1 * NN:2 * NN] * (e1 / den) +
                    y[2 * NN:3 * NN] * (e2 / den) +
                    y[3 * NN:4 * NN] * (e3 / den))


@functools.partial(jax.jit, static_argnames=())
def kernel(features, sparse, c_param, W_ac1, b_ac1, W_ca1, b_ca1,
           W_ac2, b_ac2, W_ca2, b_ca2, bn1_g, bn1_b, bn2_g, bn2_b, W_out):
    f32 = jnp.float32

    xn = pl.pallas_call(
        _norm_kernel,
        out_shape=jax.ShapeDtypeStruct((BS, F0), f32),
    )(features)

    sig = jax.nn.sigmoid(sparse[:, 0])
    thr = jnp.stack([sig[1], sig[5], sig[8], sig[10],
                     jnp.float32(0.0), sig[4], sig[7], sig[9]])[None, :]

    eye_b = jnp.eye(NN, dtype=jnp.bfloat16)
    noteye_b = jnp.float32(1.0).astype(jnp.bfloat16) - eye_b

    adj, parts_r, parts_c = pl.pallas_call(
        _adj_kernel,
        grid=(4, 4),
        in_specs=[
            pl.BlockSpec(memory_space=pltpu.SMEM),
            pl.BlockSpec((NN, F0), lambda i, j: (i, 0)),
            pl.BlockSpec((NN, F0),
                         lambda i, j: (jnp.where(i > j, i - 1, 3), 0)),
            pl.BlockSpec((NN, F0),
                         lambda i, j: (jnp.where(i > j, 3,
                                                 jnp.maximum(j - 1, 0)), 0)),
            pl.BlockSpec((NN, NN), lambda i, j: (0, 0)),
            pl.BlockSpec((NN, NN), lambda i, j: (0, 0)),
        ],
        out_specs=[
            pl.BlockSpec((NN, NN), lambda i, j: (i, j)),
            pl.BlockSpec((1, NN, 1), lambda i, j: (j, i, 0)),
            pl.BlockSpec((1, NN, 1), lambda i, j: (i, j, 0)),
        ],
        out_shape=[
            jax.ShapeDtypeStruct((BS, BS), jnp.int8),
            jax.ShapeDtypeStruct((4, BS, 1), f32),
            jax.ShapeDtypeStruct((4, BS, 1), f32),
        ],
    )(thr, xn, xn, xn, eye_b, noteye_b)

    n_r, n_c, sc_r, sc_c = pl.pallas_call(
        _degsum_kernel,
        out_shape=[jax.ShapeDtypeStruct((BS, 1), f32)] * 4,
    )(parts_r, parts_c)

    def layer_specs(feat):
        return [
            pl.BlockSpec((BS, NN), lambda d: (0, d)),
            pl.BlockSpec((NN, BS), lambda d: (d, 0)),
        ]

    b2 = lambda a: a[None, :]

    h1, ss1, sq1 = pl.pallas_call(
        _l1_kernel,
        grid=(4,),
        in_specs=layer_specs(F0) + [
            pl.BlockSpec((BS, F0), lambda d: (0, 0)),
            pl.BlockSpec((NN, F0), lambda d: (d, 0)),
            pl.BlockSpec((BS, 1), lambda d: (0, 0)),
            pl.BlockSpec((BS, 1), lambda d: (0, 0)),
            pl.BlockSpec((NN, 1), lambda d: (d, 0)),
            pl.BlockSpec((NN, 1), lambda d: (d, 0)),
            pl.BlockSpec((F0, H), lambda d: (0, 0)),
            pl.BlockSpec((1, H), lambda d: (0, 0)),
            pl.BlockSpec((F0, H), lambda d: (0, 0)),
            pl.BlockSpec((1, H), lambda d: (0, 0)),
        ],
        out_specs=[
            pl.BlockSpec((NN, 2 * H), lambda d: (d, 0)),
            pl.BlockSpec((1, 1, 2 * H), lambda d: (d, 0, 0)),
            pl.BlockSpec((1, 1, 2 * H), lambda d: (d, 0, 0)),
        ],
        out_shape=[
            jax.ShapeDtypeStruct((BS, 2 * H), f32),
            jax.ShapeDtypeStruct((4, 1, 2 * H), f32),
            jax.ShapeDtypeStruct((4, 1, 2 * H), f32),
        ],
    )(adj, adj, features, features, n_r, n_c, sc_r, sc_c,
      W_ac1, b2(b_ac1), W_ca1, b2(b_ca1))

    h2, ss2, sq2 = pl.pallas_call(
        _l2_kernel,
        grid=(4,),
        in_specs=layer_specs(2 * H) + [
            pl.BlockSpec((BS, 2 * H), lambda d: (0, 0)),
            pl.BlockSpec((NN, 2 * H), lambda d: (d, 0)),
            pl.BlockSpec((4, 1, 2 * H), lambda d: (0, 0, 0)),
            pl.BlockSpec((4, 1, 2 * H), lambda d: (0, 0, 0)),
            pl.BlockSpec((1, 2 * H), lambda d: (0, 0)),
            pl.BlockSpec((1, 2 * H), lambda d: (0, 0)),
            pl.BlockSpec((BS, 1), lambda d: (0, 0)),
            pl.BlockSpec((BS, 1), lambda d: (0, 0)),
            pl.BlockSpec((NN, 1), lambda d: (d, 0)),
            pl.BlockSpec((NN, 1), lambda d: (d, 0)),
            pl.BlockSpec((2 * H, H), lambda d: (0, 0)),
            pl.BlockSpec((1, H), lambda d: (0, 0)),
            pl.BlockSpec((2 * H, H), lambda d: (0, 0)),
            pl.BlockSpec((1, H), lambda d: (0, 0)),
        ],
        out_specs=[
            pl.BlockSpec((NN, 2 * H), lambda d: (d, 0)),
            pl.BlockSpec((1, 1, 2 * H), lambda d: (d, 0, 0)),
            pl.BlockSpec((1, 1, 2 * H), lambda d: (d, 0, 0)),
        ],
        out_shape=[
            jax.ShapeDtypeStruct((BS, 2 * H), f32),
            jax.ShapeDtypeStruct((4, 1, 2 * H), f32),
            jax.ShapeDtypeStruct((4, 1, 2 * H), f32),
        ],
    )(adj, adj, h1, h1, ss1, sq1, b2(bn1_g), b2(bn1_b),
      n_r, n_c, sc_r, sc_c,
      W_ac2, b2(b_ac2), W_ca2, b2(b_ca2))

    out = pl.pallas_call(
        _out_kernel,
        in_specs=[
            pl.BlockSpec(memory_space=pltpu.SMEM),
            pl.BlockSpec((BS, 2 * H), lambda: (0, 0)),
            pl.BlockSpec((4, 1, 2 * H), lambda: (0, 0, 0)),
            pl.BlockSpec((4, 1, 2 * H), lambda: (0, 0, 0)),
            pl.BlockSpec((1, 2 * H), lambda: (0, 0)),
            pl.BlockSpec((1, 2 * H), lambda: (0, 0)),
            pl.BlockSpec((2 * H, NC), lambda: (0, 0)),
        ],
        out_shape=jax.ShapeDtypeStruct((NN, NC), f32),
    )(c_param, h2, ss2, sq2, b2(bn2_g), b2(bn2_b), W_out)

    return out
